# baseline (speedup 1.0000x reference)
import functools

import jax
import jax.numpy as jnp
from jax import lax
from jax.experimental import pallas as pl
from jax.experimental.pallas import tpu as pltpu
from jax.experimental.pallas import tpu_sc as plsc

D_MODEL = 64
SCALE = 8.0
NCORES = 2
NSUB = 16
NW = NCORES * NSUB
SUB = 128
CHUNK = 256
RSUB = CHUNK // SUB
LANES = 16
TBLK = 1024


def _tr_body(a_ref, o_ref):
    a = a_ref[...]
    left = jnp.transpose(a[:, :TBLK])
    right = jnp.transpose(a[:, TBLK:])
    o_ref[...] = jnp.concatenate([left, right], axis=1)


def _make_t128(table):
    vocab = table.shape[0]
    nblk = (vocab + 2 * TBLK - 1) // (2 * TBLK)
    tT = jnp.swapaxes(table, 0, 1)
    return pl.pallas_call(
        _tr_body,
        grid=(nblk,),
        in_specs=[pl.BlockSpec((D_MODEL, 2 * TBLK), lambda i: (0, i))],
        out_specs=pl.BlockSpec((TBLK, 2 * D_MODEL), lambda i: (i, 0)),
        out_shape=jax.ShapeDtypeStruct((nblk * TBLK, 2 * D_MODEL),
                                       jnp.float32),
    )(tT)


def _pair_row(seg):
    return jnp.bitwise_or(
        lax.shift_left(lax.shift_right_logical(seg, 11), 10),
        jnp.bitwise_and(seg, 1023))


def _fire_gathers(t128_hbm, idx_v, h_buf, gbuf, gsem, g):
    for rr in range(RSUB):
        for s in range(SUB // LANES):
            sl = pl.ds(s * LANES, LANES)
            h_buf[rr, sl] = _pair_row(idx_v[g * RSUB + rr, sl])
    for rr in range(RSUB):
        pltpu.async_copy(
            t128_hbm.at[h_buf.at[rr]],
            gbuf.at[pl.ds(rr * SUB, SUB)],
            gsem,
        )


def _wait_gathers(out_hbm, gbuf, gsem):
    pltpu.make_async_copy(out_hbm.at[pl.ds(0, CHUNK)], gbuf, gsem).wait()


def _scale_chunk(idx_v, gbuf, obuf, g):
    iota = lax.iota(jnp.int32, LANES)
    for rr in range(RSUB):
        def group_body(jj, c, rr=rr):
            r0 = rr * SUB
            seg = idx_v[g * RSUB + rr, pl.ds(jj * LANES, LANES)]
            p64 = jnp.bitwise_and(lax.shift_right_logical(seg, 4), 64)
            rows16 = r0 + jj * LANES + iota
            for k in range(D_MODEL):
                colv = jnp.bitwise_or(p64, k)
                v = plsc.load_gather(gbuf, [rows16, colv])
                plsc.store_scatter(
                    obuf, [rows16, jnp.full((LANES,), k, jnp.int32)],
                    v * SCALE)
            return c
        lax.fori_loop(0, SUB // LANES, group_body, 0)


def _emb_body(x_hbm, t128_hbm, out_hbm,
              idx_v, gbuf0, gbuf1, obuf0, obuf1, h0, h1,
              gsem0, gsem1, osem0, osem1):
    nrows_w = x_hbm.shape[0] // NW
    rows_per_w = nrows_w * SUB
    nchunk = rows_per_w // CHUNK
    wid = lax.axis_index("s") * NCORES + lax.axis_index("c")
    base = wid * rows_per_w

    gbufs = (gbuf0, gbuf1)
    obufs = (obuf0, obuf1)
    hbufs = (h0, h1)
    gsems = (gsem0, gsem1)
    osems = (osem0, osem1)

    pltpu.sync_copy(x_hbm.at[pl.ds(wid * nrows_w, nrows_w)], idx_v)

    for b in range(2):
        _fire_gathers(t128_hbm, idx_v, hbufs[b], gbufs[b], gsems[b], b)

    def body(i, carry):
        for b in range(2):
            g = 2 * i + b
            _wait_gathers(out_hbm, gbufs[b], gsems[b])

            @pl.when(i >= 1)
            def _():
                pltpu.make_async_copy(
                    obufs[b],
                    out_hbm.at[pl.ds(base, CHUNK), pl.ds(0, D_MODEL)],
                    osems[b]).wait()

            _scale_chunk(idx_v, gbufs[b], obufs[b], g)

            @pl.when(g + 2 < nchunk)
            def _():
                _fire_gathers(t128_hbm, idx_v, hbufs[b], gbufs[b], gsems[b],
                              g + 2)

            pltpu.async_copy(
                obufs[b],
                out_hbm.at[pl.ds(base + g * CHUNK, CHUNK), pl.ds(0, D_MODEL)],
                osems[b])
        return carry

    lax.fori_loop(0, nchunk // 2, body, 0)

    for b in range(2):
        pltpu.make_async_copy(
            obufs[b],
            out_hbm.at[pl.ds(base, CHUNK), pl.ds(0, D_MODEL)],
            osems[b]).wait()


@functools.partial(jax.jit, static_argnames=())
def kernel(x, table):
    b_total = x.size
    xf = x.reshape(b_total // SUB, SUB)
    t128 = _make_t128(table)
    mesh = plsc.VectorSubcoreMesh(core_axis_name="c", subcore_axis_name="s")
    nrows_w = xf.shape[0] // NW
    run = pl.kernel(
        _emb_body,
        mesh=mesh,
        out_type=jax.ShapeDtypeStruct((b_total, 2 * D_MODEL), jnp.float32),
        scratch_types=[
            pltpu.VMEM((nrows_w, SUB), jnp.int32),
            pltpu.VMEM((CHUNK, 2 * D_MODEL), jnp.float32),
            pltpu.VMEM((CHUNK, 2 * D_MODEL), jnp.float32),
            pltpu.VMEM((CHUNK, D_MODEL), jnp.float32),
            pltpu.VMEM((CHUNK, D_MODEL), jnp.float32),
            pltpu.VMEM((RSUB, SUB), jnp.int32),
            pltpu.VMEM((RSUB, SUB), jnp.int32),
            pltpu.SemaphoreType.DMA,
            pltpu.SemaphoreType.DMA,
            pltpu.SemaphoreType.DMA,
            pltpu.SemaphoreType.DMA,
        ],
        compiler_params=pltpu.CompilerParams(
            use_tc_tiling_on_sc=False, needs_layout_passes=False),
    )
    out = run(xf, t128)
    return out[:, :D_MODEL].reshape(x.shape + (D_MODEL,))


# speedup vs baseline: 2.2176x; 2.2176x over previous
import functools

import jax
import jax.numpy as jnp
from jax import lax
from jax.experimental import pallas as pl
from jax.experimental.pallas import tpu as pltpu
from jax.experimental.pallas import tpu_sc as plsc

D_MODEL = 64
SCALE = 8.0
NCORES = 2
NSUB = 16
NW = NCORES * NSUB
SUB = 128
CHUNK = 256
RSUB = CHUNK // SUB
LANES = 16
RU = 8


def _fire_gathers(table_hbm, idx_v, gbuf, gsem, g):
    for j in range(RSUB):
        pltpu.async_copy(
            table_hbm.at[idx_v.at[g * RSUB + j]],
            gbuf.at[pl.ds(j * SUB, SUB)],
            gsem,
        )


def _wait_gathers(out_hbm, gbuf, gsem):
    pltpu.make_async_copy(
        out_hbm.at[pl.ds(0, CHUNK)], gbuf, gsem).wait()


def _scale_chunk(gbuf, obuf):
    def row_body(i, c):
        r0 = i * RU
        for u in range(RU):
            for col in range(D_MODEL // LANES):
                sl = pl.ds(col * LANES, LANES)
                obuf[r0 + u, sl] = gbuf[r0 + u, sl] * SCALE
        return c
    lax.fori_loop(0, CHUNK // RU, row_body, 0)


def _emb_body(x_hbm, table_hbm, out_hbm,
              idx_v, gbuf0, gbuf1, obuf0, obuf1,
              gsem0, gsem1, osem0, osem1):
    nrows_w = x_hbm.shape[0] // NW
    rows_per_w = nrows_w * SUB
    nchunk = rows_per_w // CHUNK
    wid = lax.axis_index("s") * NCORES + lax.axis_index("c")
    base = wid * rows_per_w

    gbufs = (gbuf0, gbuf1)
    obufs = (obuf0, obuf1)
    gsems = (gsem0, gsem1)
    osems = (osem0, osem1)

    pltpu.sync_copy(x_hbm.at[pl.ds(wid * nrows_w, nrows_w)], idx_v)

    for b in range(2):
        _fire_gathers(table_hbm, idx_v, gbufs[b], gsems[b], b)

    def body(i, carry):
        for b in range(2):
            g = 2 * i + b
            _wait_gathers(out_hbm, gbufs[b], gsems[b])

            @pl.when(i >= 1)
            def _():
                pltpu.make_async_copy(
                    obufs[b],
                    out_hbm.at[pl.ds(base, CHUNK), pl.ds(0, D_MODEL)],
                    osems[b]).wait()

            _scale_chunk(gbufs[b], obufs[b])

            @pl.when(g + 2 < nchunk)
            def _():
                _fire_gathers(table_hbm, idx_v, gbufs[b], gsems[b], g + 2)

            pltpu.async_copy(
                obufs[b],
                out_hbm.at[pl.ds(base + g * CHUNK, CHUNK), pl.ds(0, D_MODEL)],
                osems[b])
        return carry

    lax.fori_loop(0, nchunk // 2, body, 0)

    for b in range(2):
        pltpu.make_async_copy(
            obufs[b],
            out_hbm.at[pl.ds(base, CHUNK), pl.ds(0, D_MODEL)],
            osems[b]).wait()


@functools.partial(jax.jit, static_argnames=())
def kernel(x, table):
    b_total = x.size
    xf = x.reshape(b_total // SUB, SUB)
    mesh = plsc.VectorSubcoreMesh(core_axis_name="c", subcore_axis_name="s")
    nrows_w = xf.shape[0] // NW
    run = pl.kernel(
        _emb_body,
        mesh=mesh,
        out_type=jax.ShapeDtypeStruct((b_total, 2 * D_MODEL), jnp.float32),
        scratch_types=[
            pltpu.VMEM((nrows_w, SUB), jnp.int32),
            pltpu.VMEM((CHUNK, 2 * D_MODEL), jnp.float32),
            pltpu.VMEM((CHUNK, 2 * D_MODEL), jnp.float32),
            pltpu.VMEM((CHUNK, D_MODEL), jnp.float32),
            pltpu.VMEM((CHUNK, D_MODEL), jnp.float32),
            pltpu.SemaphoreType.DMA,
            pltpu.SemaphoreType.DMA,
            pltpu.SemaphoreType.DMA,
            pltpu.SemaphoreType.DMA,
        ],
        compiler_params=pltpu.CompilerParams(use_tc_tiling_on_sc=False),
    )
    out = run(xf, jnp.concatenate(
        [table, jnp.zeros_like(table)], axis=1))
    return out[:, :D_MODEL].reshape(x.shape + (D_MODEL,))


# speedup vs baseline: 2.8464x; 1.2836x over previous
import functools

import jax
import jax.numpy as jnp
from jax import lax
from jax.experimental import pallas as pl
from jax.experimental.pallas import tpu as pltpu
from jax.experimental.pallas import tpu_sc as plsc

D_MODEL = 64
SCALE = 8.0
NCORES = 2
NSUB = 16
NW = NCORES * NSUB
SUB = 128
CHUNK = 256
RSUB = CHUNK // SUB
LANES = 16
RU = 8


def _fire_gathers(table_hbm, idx_v, gbuf, gsem, g):
    for j in range(RSUB):
        pltpu.async_copy(
            table_hbm.at[idx_v.at[g * RSUB + j]],
            gbuf.at[pl.ds(j * SUB, SUB)],
            gsem,
        )


def _wait_gathers(out_hbm, gbuf, gsem):
    pltpu.make_async_copy(
        out_hbm.at[pl.ds(0, CHUNK), pl.ds(0, D_MODEL)], gbuf, gsem).wait()


def _scale_chunk(gbuf, obuf):
    def row_body(i, c):
        r0 = i * RU
        for u in range(RU):
            for col in range(D_MODEL // LANES):
                sl = pl.ds(col * LANES, LANES)
                obuf[r0 + u, sl] = gbuf[r0 + u, sl] * SCALE
        return c
    lax.fori_loop(0, CHUNK // RU, row_body, 0)


def _emb_body(x_hbm, table_hbm, out_hbm,
              idx_v, gbuf0, gbuf1, obuf0, obuf1,
              gsem0, gsem1, osem0, osem1):
    nrows_w = x_hbm.shape[0] // NW
    rows_per_w = nrows_w * SUB
    nchunk = rows_per_w // CHUNK
    wid = lax.axis_index("s") * NCORES + lax.axis_index("c")
    base = wid * rows_per_w

    gbufs = (gbuf0, gbuf1)
    obufs = (obuf0, obuf1)
    gsems = (gsem0, gsem1)
    osems = (osem0, osem1)

    pltpu.sync_copy(x_hbm.at[pl.ds(wid * nrows_w, nrows_w)], idx_v)

    for b in range(2):
        _fire_gathers(table_hbm, idx_v, gbufs[b], gsems[b], b)

    def body(i, carry):
        for b in range(2):
            g = 2 * i + b
            _wait_gathers(out_hbm, gbufs[b], gsems[b])

            @pl.when(i >= 1)
            def _():
                pltpu.make_async_copy(
                    obufs[b],
                    out_hbm.at[pl.ds(base, CHUNK), pl.ds(0, D_MODEL)],
                    osems[b]).wait()

            _scale_chunk(gbufs[b], obufs[b])

            @pl.when(g + 2 < nchunk)
            def _():
                _fire_gathers(table_hbm, idx_v, gbufs[b], gsems[b], g + 2)

            pltpu.async_copy(
                obufs[b],
                out_hbm.at[pl.ds(base + g * CHUNK, CHUNK), pl.ds(0, D_MODEL)],
                osems[b])
        return carry

    lax.fori_loop(0, nchunk // 2, body, 0)

    for b in range(2):
        pltpu.make_async_copy(
            obufs[b],
            out_hbm.at[pl.ds(base, CHUNK), pl.ds(0, D_MODEL)],
            osems[b]).wait()


@functools.partial(jax.jit, static_argnames=())
def kernel(x, table):
    b_total = x.size
    xf = x.reshape(b_total // SUB, SUB)
    mesh = plsc.VectorSubcoreMesh(core_axis_name="c", subcore_axis_name="s")
    nrows_w = xf.shape[0] // NW
    run = pl.kernel(
        _emb_body,
        mesh=mesh,
        out_type=jax.ShapeDtypeStruct((b_total, 2 * D_MODEL), jnp.float32),
        scratch_types=[
            pltpu.VMEM((nrows_w, SUB), jnp.int32),
            pltpu.VMEM((CHUNK, D_MODEL), jnp.float32),
            pltpu.VMEM((CHUNK, D_MODEL), jnp.float32),
            pltpu.VMEM((CHUNK, D_MODEL), jnp.float32),
            pltpu.VMEM((CHUNK, D_MODEL), jnp.float32),
            pltpu.SemaphoreType.DMA,
            pltpu.SemaphoreType.DMA,
            pltpu.SemaphoreType.DMA,
            pltpu.SemaphoreType.DMA,
        ],
        compiler_params=pltpu.CompilerParams(use_tc_tiling_on_sc=False),
    )
    out = run(xf, table)
    return out[:, :D_MODEL].reshape(x.shape + (D_MODEL,))
